# baseline (speedup 1.0000x reference)
import functools

import jax
import jax.numpy as jnp
from jax import lax
from jax.experimental import pallas as pl
from jax.experimental.pallas import tpu as pltpu
from jax.experimental.pallas import tpu_sc as plsc

_NUM_EMBEDDINGS = 8192
_DIM = 256
_ROWS = 4096

_RB = 4096
_CB = 4096
_NR = _ROWS // _RB
_NC = _NUM_EMBEDDINGS // _CB
_MM = 512


def _vq_body(x_full, e_full, xn_full, out_et, out_idx, e2s, ens, rm_s, ri_s):
    c = pl.program_id(0)
    r = pl.program_id(1)

    @pl.when((c == 0) & (r == 0))
    def _derive():
        e = e_full[...]
        e2s[...] = -2.0 * e
        ens[...] = jnp.sum(e * e, axis=0, keepdims=True)

    @pl.when(r == 0)
    def _transpose():
        out_et[...] = e_full[:, pl.ds(c * _CB, _CB)].T

    rsl = pl.ds(r * _RB, _RB)

    @pl.when(c == 0)
    def _init():
        rm_s[rsl, :] = jnp.full((_RB, 128), jnp.inf, dtype=jnp.float32)
        ri_s[rsl, :] = jnp.zeros((_RB, 128), dtype=jnp.int32)

    xb = x_full[rsl, :]
    xn_v = xn_full[rsl, :]
    rm = rm_s[rsl, :]
    ri = ri_s[rsl, :]
    for km in range(_CB // _MM):
        s2 = jnp.dot(
            xb,
            e2s[:, pl.ds(c * _CB + km * _MM, _MM)],
            preferred_element_type=jnp.float32,
        )
        for kk in range(_MM // 128):
            k = km * (_MM // 128) + kk
            en_sl = ens[:, pl.ds(c * _CB + k * 128, 128)]
            dk = (xn_v + en_sl) + s2[:, kk * 128 : (kk + 1) * 128]
            upd = dk < rm
            rm = jnp.where(upd, dk, rm)
            ri = jnp.where(
                upd, jnp.full((_RB, 128), c * (_CB // 128) + k, jnp.int32), ri
            )
    rm_s[rsl, :] = rm
    ri_s[rsl, :] = ri

    @pl.when(c == pl.num_programs(0) - 1)
    def _extract():
        lane = lax.broadcasted_iota(jnp.int32, (_RB, 128), 1)
        idx = ri * 128 + lane
        m = jnp.min(rm, axis=1, keepdims=True)
        cand = jnp.where(rm == m, idx, jnp.int32(2**30))
        out_idx[...] = jnp.min(cand, axis=1, keepdims=True)


def _tc_vq(flattened, embeddings, x_norm):
    out_et, out_idx = pl.pallas_call(
        _vq_body,
        grid=(_NC, _NR),
        in_specs=[
            pl.BlockSpec((_ROWS, _DIM), lambda c, r: (0, 0)),
            pl.BlockSpec((_DIM, _NUM_EMBEDDINGS), lambda c, r: (0, 0)),
            pl.BlockSpec((_ROWS, 1), lambda c, r: (0, 0)),
        ],
        out_specs=[
            pl.BlockSpec((_CB, _DIM), lambda c, r: (c, 0)),
            pl.BlockSpec((_RB, 1), lambda c, r: (r, 0)),
        ],
        out_shape=[
            jax.ShapeDtypeStruct((_NUM_EMBEDDINGS, _DIM), jnp.float32),
            jax.ShapeDtypeStruct((_ROWS, 1), jnp.int32),
        ],
        scratch_shapes=[
            pltpu.VMEM((_DIM, _NUM_EMBEDDINGS), jnp.float32),
            pltpu.VMEM((1, _NUM_EMBEDDINGS), jnp.float32),
            pltpu.VMEM((_ROWS, 128), jnp.float32),
            pltpu.VMEM((_ROWS, 128), jnp.int32),
        ],
    )(flattened, embeddings, x_norm)
    return out_et, out_idx.reshape(_ROWS)


def _sc_gather(table, idx):
    info = plsc.get_sparse_core_info()
    ncores, nsub = info.num_cores, info.num_subcores
    nw = ncores * nsub
    b_per_w = _ROWS // nw
    mesh = plsc.VectorSubcoreMesh(core_axis_name="c", subcore_axis_name="s")

    @functools.partial(
        pl.kernel,
        mesh=mesh,
        out_type=jax.ShapeDtypeStruct((_ROWS, _DIM), jnp.float32),
        scratch_types=[
            pltpu.VMEM((b_per_w,), jnp.int32),
            pltpu.VMEM((b_per_w, _DIM), jnp.float32),
            pltpu.SemaphoreType.DMA,
        ],
    )
    def gk(table_hbm, idx_hbm, out_hbm, idx_v, rows_v, sem):
        wid = lax.axis_index("s") * ncores + lax.axis_index("c")
        base = wid * b_per_w
        pltpu.sync_copy(idx_hbm.at[pl.ds(base, b_per_w)], idx_v)
        pltpu.async_copy(table_hbm.at[idx_v], rows_v, sem).wait()
        pltpu.sync_copy(rows_v, out_hbm.at[pl.ds(base, b_per_w)])

    return gk(table, idx)


def kernel(x, embeddings):
    input_shape = x.shape
    flattened = jnp.reshape(x, (-1, _DIM))
    x_norm = jnp.sum(flattened**2, axis=1, keepdims=True)

    emb_t, idx = _tc_vq(flattened, embeddings, x_norm)

    quantized = _sc_gather(emb_t, idx)
    return (jnp.reshape(quantized, input_shape), idx)
